# Initial kernel scaffold; baseline (speedup 1.0000x reference)
import jax
import jax.numpy as jnp
from jax.experimental import pallas as pl


def kernel(x, embed_table):
    raise NotImplementedError("write your pallas kernel here")


# speedup vs baseline: 2.4565x; 2.4565x over previous
import functools

import jax
import jax.numpy as jnp
import numpy as np
from jax import lax
from jax.experimental import pallas as pl
from jax.experimental.pallas import tpu as pltpu
from jax.experimental.pallas import tpu_sc as plsc

VOCAB = 100000
D_MODEL = 128
MAX_LEN = 512
B = 1024
L = 200

NUM_CORES = 2
NUM_SUBCORES = 16
NW = NUM_CORES * NUM_SUBCORES

N_ROWS = B * L
ROWS_PER_W = N_ROWS // NW
CHUNK = 80
CHUNKS_PER_W = ROWS_PER_W // CHUNK
NBUF = 4


def _pos_encoding() -> np.ndarray:
    pos = np.arange(MAX_LEN, dtype=np.float64)[:, None]
    i = np.arange(0, D_MODEL, 2, dtype=np.float64)[None, :]
    loc = pos / (10000.0 ** (i / D_MODEL))
    enc = np.zeros((MAX_LEN, D_MODEL), dtype=np.float32)
    enc[:, 0::2] = np.sin(loc)
    enc[:, 1::2] = np.cos(loc)
    return np.concatenate([enc[:L], enc[:L]], axis=0)


_ENC2 = _pos_encoding()


def _sc_kernel():
    mesh = plsc.VectorSubcoreMesh(core_axis_name="c", subcore_axis_name="s")

    @functools.partial(
        pl.kernel,
        mesh=mesh,
        out_type=jax.ShapeDtypeStruct((N_ROWS, D_MODEL), jnp.float32),
        scratch_types=[
            pltpu.VMEM((CHUNKS_PER_W, CHUNK), jnp.int32),
            pltpu.VMEM((2 * L, D_MODEL), jnp.float32),
        ]
        + [pltpu.VMEM((CHUNK, D_MODEL), jnp.float32) for _ in range(NBUF)]
        + [pltpu.SemaphoreType.DMA for _ in range(2 * NBUF)],
    )
    def k(table_hbm, xr_hbm, enc_hbm, out_hbm, idx_v, enc_v,
          b0, b1, b2, b3, g0, g1, g2, g3, s0, s1, s2, s3):
        bufs = (b0, b1, b2, b3)
        gsems = (g0, g1, g2, g3)
        ssems = (s0, s1, s2, s3)
        wid = lax.axis_index("s") * NUM_CORES + lax.axis_index("c")
        crow0 = wid * CHUNKS_PER_W
        row0 = wid * ROWS_PER_W

        pltpu.sync_copy(xr_hbm.at[pl.ds(crow0, CHUNKS_PER_W)], idx_v)
        pltpu.sync_copy(enc_hbm, enc_v)

        def start_gather(c, b):
            pltpu.async_copy(table_hbm.at[idx_v.at[c]], bufs[b], gsems[b])

        def start_store(c, b):
            pltpu.async_copy(bufs[b],
                             out_hbm.at[pl.ds(row0 + c * CHUNK, CHUNK)],
                             ssems[b])

        def wait_gather(c, b):
            pltpu.make_async_copy(table_hbm.at[idx_v.at[c]],
                                  bufs[b], gsems[b]).wait()

        def wait_store(c, b):
            pltpu.make_async_copy(bufs[b],
                                  out_hbm.at[pl.ds(row0 + c * CHUNK, CHUNK)],
                                  ssems[b]).wait()

        def add_enc(c, b):
            base = lax.rem(c * CHUNK, L)

            def row_body(r, _):
                for col in range(D_MODEL // 16):
                    sl = pl.ds(col * 16, 16)
                    bufs[b][r, sl] = bufs[b][r, sl] + enc_v[base + r, sl]
                return 0

            lax.fori_loop(0, CHUNK, row_body, 0, unroll=2)

        for b in range(NBUF):
            start_gather(b, b)

        def outer(i, _):
            c0 = i * NBUF
            for b in range(NBUF):
                c = c0 + b
                wait_gather(c, b)
                add_enc(c, b)
                start_store(c, b)
            for b in range(NBUF):
                c = c0 + b
                wait_store(c, b)
                start_gather(c + NBUF, b)
            return 0

        n_main = CHUNKS_PER_W // NBUF - 1
        lax.fori_loop(0, n_main, outer, 0)

        for b in range(NBUF):
            c = n_main * NBUF + b
            wait_gather(c, b)
            add_enc(c, b)
            start_store(c, b)
        for b in range(NBUF):
            c = n_main * NBUF + b
            wait_store(c, b)

    return k


_K = _sc_kernel()


def kernel(x, embed_table):
    xr = jnp.asarray(x, jnp.int32).reshape(N_ROWS // CHUNK, CHUNK)
    enc = jnp.asarray(_ENC2)
    out = _K(embed_table, xr, enc)
    return out.reshape(B, L, D_MODEL)


# speedup vs baseline: 5.5867x; 2.2742x over previous
import functools

import jax
import jax.numpy as jnp
import numpy as np
from jax import lax
from jax.experimental import pallas as pl
from jax.experimental.pallas import tpu as pltpu
from jax.experimental.pallas import tpu_sc as plsc

VOCAB = 100000
D_MODEL = 128
MAX_LEN = 512
B = 1024
L = 200

NUM_CORES = 2
NUM_SUBCORES = 16
NW = NUM_CORES * NUM_SUBCORES

N_ROWS = B * L
ROWS_PER_W = N_ROWS // NW
CHUNK = 80
CHUNKS_PER_W = ROWS_PER_W // CHUNK
NBUF = 4


def _pos_encoding() -> np.ndarray:
    pos = np.arange(MAX_LEN, dtype=np.float64)[:, None]
    i = np.arange(0, D_MODEL, 2, dtype=np.float64)[None, :]
    loc = pos / (10000.0 ** (i / D_MODEL))
    enc = np.zeros((MAX_LEN, D_MODEL), dtype=np.float32)
    enc[:, 0::2] = np.sin(loc)
    enc[:, 1::2] = np.cos(loc)
    return np.concatenate([enc[:L], enc[:L]], axis=0)


_ENC2 = _pos_encoding()


def _sc_kernel():
    mesh = plsc.VectorSubcoreMesh(core_axis_name="c", subcore_axis_name="s")

    @functools.partial(
        pl.kernel,
        mesh=mesh,
        out_type=jax.ShapeDtypeStruct((N_ROWS, D_MODEL), jnp.float32),
        scratch_types=[
            pltpu.VMEM((CHUNKS_PER_W, CHUNK), jnp.int32),
            pltpu.VMEM((2 * L, D_MODEL), jnp.float32),
        ]
        + [pltpu.VMEM((CHUNK, D_MODEL), jnp.float32) for _ in range(NBUF)]
        + [pltpu.SemaphoreType.DMA for _ in range(2 * NBUF)],
    )
    def k(table_hbm, xr_hbm, enc_hbm, out_hbm, idx_v, enc_v,
          b0, b1, b2, b3, g0, g1, g2, g3, s0, s1, s2, s3):
        bufs = (b0, b1, b2, b3)
        gsems = (g0, g1, g2, g3)
        ssems = (s0, s1, s2, s3)
        wid = lax.axis_index("s") * NUM_CORES + lax.axis_index("c")
        crow0 = wid * CHUNKS_PER_W
        row0 = wid * ROWS_PER_W

        pltpu.sync_copy(xr_hbm.at[pl.ds(crow0, CHUNKS_PER_W)], idx_v)
        pltpu.sync_copy(enc_hbm, enc_v)

        def start_gather(c, b):
            pltpu.async_copy(table_hbm.at[idx_v.at[c]], bufs[b], gsems[b])

        def start_store(c, b):
            pltpu.async_copy(bufs[b],
                             out_hbm.at[pl.ds(row0 + c * CHUNK, CHUNK)],
                             ssems[b])

        def wait_gather(c, b):
            pltpu.make_async_copy(table_hbm.at[idx_v.at[c]],
                                  bufs[b], gsems[b]).wait()

        def wait_store(c, b):
            pltpu.make_async_copy(bufs[b],
                                  out_hbm.at[pl.ds(row0 + c * CHUNK, CHUNK)],
                                  ssems[b]).wait()

        def add_enc(c, b):
            base = lax.rem(c * CHUNK, L)

            def row_body(r, _):
                cols = range(D_MODEL // 16)
                g = [bufs[b][r, pl.ds(col * 16, 16)] for col in cols]
                e = [enc_v[base + r, pl.ds(col * 16, 16)] for col in cols]
                for col in cols:
                    bufs[b][r, pl.ds(col * 16, 16)] = g[col] + e[col]
                return 0

            lax.fori_loop(0, CHUNK, row_body, 0, unroll=2)

        for b in range(NBUF):
            start_gather(b, b)

        def outer(i, _):
            c0 = i * NBUF
            for b in range(NBUF):
                c = c0 + b
                wait_gather(c, b)
                add_enc(c, b)
                start_store(c, b)
            for b in range(NBUF):
                c = c0 + b
                wait_store(c, b)
                start_gather(c + NBUF, b)
            return 0

        n_main = CHUNKS_PER_W // NBUF - 1
        lax.fori_loop(0, n_main, outer, 0)

        for b in range(NBUF):
            c = n_main * NBUF + b
            wait_gather(c, b)
            add_enc(c, b)
            start_store(c, b)
        for b in range(NBUF):
            c = n_main * NBUF + b
            wait_store(c, b)

    return k


_K = _sc_kernel()


def kernel(x, embed_table):
    xr = jnp.asarray(x, jnp.int32).reshape(N_ROWS // CHUNK, CHUNK)
    enc = jnp.asarray(_ENC2)
    out = _K(embed_table, xr, enc)
    return out.reshape(B, L, D_MODEL)
